# baseline (speedup 1.0000x reference)
import functools

import jax
import jax.numpy as jnp
from jax import lax
from jax.experimental import pallas as pl
from jax.experimental.pallas import tpu as pltpu
from jax.experimental.pallas import tpu_sc as plsc

N = 10000
E = 320000
D = 128
H = 128
L = 4
C = 32
G = 256

NC = 2
NS = 16
EPC = E // NC
EPT = EPC // NS
K = 128
NFULL = EPT // K
TAIL = EPT - NFULL * K
ZB = 624
ZR = 16
ZCNT = ZB // ZR

CH_TOT = 2560
E_PAD = CH_TOT * K
CPT = CH_TOT // (NC * NS)
NB = 5
NGRP = CPT // NB

PR_T = (N // NC) // NS
PK = 104
PNC = PR_T // PK
PREM = N // NC - NS * PR_T
GPT = G // NS


def _zero_vmem(ref, rows):
    zero16 = jnp.zeros((16,), jnp.float32)
    for r in range(rows):
        for q in range(D // 16):
            ref[r, pl.ds(q * 16, 16)] = zero16


def _fill_ones(ref, rows):
    one16 = jnp.full((16,), 1.0, jnp.float32)
    for r in range(rows):
        for q in range(D // 16):
            ref[r, pl.ds(q * 16, 16)] = one16


ZT = 16


def _zero_acc(acc, zbuf, s):
    rbase = s * ZB

    @pl.loop(0, ZCNT)
    def _(i):
        pltpu.sync_copy(zbuf, acc.at[pl.ds(rbase + i * ZR, ZR), :])

    @pl.when(s == NS - 1)
    def _():
        pltpu.sync_copy(zbuf.at[pl.ds(0, ZT), :], acc.at[pl.ds(N - ZT, ZT), :])


def _write_acc(acc, out_hbm, c, s):
    rbase = s * ZB
    pltpu.sync_copy(acc.at[pl.ds(rbase, ZB), :],
                    out_hbm.at[pl.ds(c * N + rbase, ZB), :])

    @pl.when(s == NS - 1)
    def _():
        pltpu.sync_copy(acc.at[pl.ds(N - ZT, ZT), :],
                        out_hbm.at[pl.ds(c * N + N - ZT, ZT), :])


def _agg_body(h_hbm, src_hbm, dst2_hbm, out_hbm,
              acc, didx2, r0, r1, sb0, sb1, zbuf,
              g0, g1, i0, i1):
    c = lax.axis_index("c")
    s = lax.axis_index("s")
    rs = (r0, r1)
    sbs = (sb0, sb1)
    gsems = (g0, g1)
    isems = (i0, i1)

    _zero_vmem(zbuf, ZR)
    _zero_acc(acc, zbuf, s)

    cbase = (c * NS + s) * CPT
    pltpu.sync_copy(dst2_hbm.at[pl.ds(cbase, CPT), :], didx2)

    plsc.subcore_barrier()

    def drain_g(b):
        pltpu.make_async_copy(h_hbm.at[sbs[b]], rs[b], gsems[b]).wait()

    def drain_i(b):
        pltpu.make_async_copy(src_hbm.at[pl.ds(0, K)], sbs[b],
                              isems[b]).wait()

    def load_idx(i, b):
        pltpu.async_copy(src_hbm.at[pl.ds((cbase + i) * K, K)], sbs[b],
                         isems[b])

    pltpu.sync_copy(src_hbm.at[pl.ds(cbase * K, K)], sb0)
    load_idx(1, 1)
    pltpu.async_copy(h_hbm.at[sb0], r0, g0)

    def step(i, b, issue_idx):
        nb = 1 - b
        drain_g(b)
        if issue_idx:
            load_idx(i + 2, b)
        drain_i(nb)
        pltpu.async_copy(h_hbm.at[sbs[nb]], rs[nb], gsems[nb])
        pltpu.sync_copy(rs[b], acc.at[didx2.at[i]], add=True)

    @pl.loop(0, (CPT - 2) // 2)
    def _(j):
        i = j * 2
        step(i, 0, True)
        step(i + 1, 1, True)

    step(CPT - 2, 0, False)
    drain_g(1)
    pltpu.sync_copy(r1, acc.at[didx2.at[CPT - 1]], add=True)

    plsc.subcore_barrier()
    _write_acc(acc, out_hbm, c, s)


@functools.lru_cache(maxsize=None)
def _make_agg():
    mesh = plsc.VectorSubcoreMesh(core_axis_name="c", subcore_axis_name="s")
    return pl.kernel(
        _agg_body,
        out_type=jax.ShapeDtypeStruct((NC * N, D), jnp.float32),
        mesh=mesh,
        scratch_types=[
            pltpu.VMEM_SHARED((N + 8, D), jnp.float32),
            pltpu.VMEM((CPT, K), jnp.int32),
            pltpu.VMEM((K, D), jnp.float32),
            pltpu.VMEM((K, D), jnp.float32),
            pltpu.VMEM((K,), jnp.int32),
            pltpu.VMEM((K,), jnp.int32),
            pltpu.VMEM((ZR, D), jnp.float32),
            pltpu.SemaphoreType.DMA,
            pltpu.SemaphoreType.DMA,
            pltpu.SemaphoreType.DMA,
            pltpu.SemaphoreType.DMA,
        ],
    )


def _deg_body(dst2_hbm, out_hbm, acc, didx2, ones_v, zbuf, sem):
    c = lax.axis_index("c")
    s = lax.axis_index("s")

    _zero_vmem(zbuf, ZR)
    _fill_ones(ones_v, K)
    _zero_acc(acc, zbuf, s)

    cbase = (c * NS + s) * CPT
    pltpu.sync_copy(dst2_hbm.at[pl.ds(cbase, CPT), :], didx2)

    plsc.subcore_barrier()

    @pl.loop(0, CPT)
    def _(i):
        pltpu.sync_copy(ones_v, acc.at[didx2.at[i]], add=True)

    plsc.subcore_barrier()
    _write_acc(acc, out_hbm, c, s)


@functools.lru_cache(maxsize=None)
def _make_deg():
    mesh = plsc.VectorSubcoreMesh(core_axis_name="c", subcore_axis_name="s")
    return pl.kernel(
        _deg_body,
        out_type=jax.ShapeDtypeStruct((NC * N, D), jnp.float32),
        mesh=mesh,
        scratch_types=[
            pltpu.VMEM_SHARED((N + 8, D), jnp.float32),
            pltpu.VMEM((CPT, K), jnp.int32),
            pltpu.VMEM((K, D), jnp.float32),
            pltpu.VMEM((ZR, D), jnp.float32),
            pltpu.SemaphoreType.DMA,
        ],
    )


def _pool_body(h0_hbm, h1_hbm, h2_hbm, h3_hbm, batch_hbm, out_hbm, cnt_hbm,
               a0, a1, a2, a3, cacc, bidx, rbuf, bidx_t, rbuf_t,
               zbuf, ones_v, sem):
    c = lax.axis_index("c")
    s = lax.axis_index("s")
    accs = (a0, a1, a2, a3)
    hs = (h0_hbm, h1_hbm, h2_hbm, h3_hbm)

    _zero_vmem(zbuf, GPT)
    _fill_ones(ones_v, PK)

    gbase = s * GPT
    for a in accs:
        pltpu.sync_copy(zbuf, a.at[pl.ds(gbase, GPT), :])
    pltpu.sync_copy(zbuf, cacc.at[pl.ds(gbase, GPT), :])
    plsc.subcore_barrier()

    nbase = c * (N // NC) + s * PR_T

    @pl.loop(0, PNC)
    def _(i):
        off = pl.multiple_of(nbase + i * PK, 8)
        pltpu.sync_copy(batch_hbm.at[pl.ds(off, PK)], bidx)
        for h_hbm, a in zip(hs, accs):
            pltpu.sync_copy(h_hbm.at[pl.ds(off, PK), :], rbuf)
            pltpu.sync_copy(rbuf, a.at[bidx], add=True)
        pltpu.sync_copy(ones_v, cacc.at[bidx], add=True)

    @pl.when(s == NS - 1)
    def _():
        off = pl.multiple_of(c * (N // NC) + NS * PR_T, 8)
        pltpu.sync_copy(batch_hbm.at[pl.ds(off, PREM)], bidx_t)
        for h_hbm, a in zip(hs, accs):
            pltpu.sync_copy(h_hbm.at[pl.ds(off, PREM), :], rbuf_t)
            pltpu.sync_copy(rbuf_t, a.at[bidx_t], add=True)
        pltpu.sync_copy(ones_v.at[pl.ds(0, PREM), :], cacc.at[bidx_t],
                        add=True)

    plsc.subcore_barrier()

    for l, a in enumerate(accs):
        obase = (c * L + l) * G + gbase
        pltpu.sync_copy(a.at[pl.ds(gbase, GPT), :],
                        out_hbm.at[pl.ds(obase, GPT), :])
    pltpu.sync_copy(cacc.at[pl.ds(gbase, GPT), :],
                    cnt_hbm.at[pl.ds(c * G + gbase, GPT), :])


@functools.lru_cache(maxsize=None)
def _make_pool():
    mesh = plsc.VectorSubcoreMesh(core_axis_name="c", subcore_axis_name="s")
    return pl.kernel(
        _pool_body,
        out_type=(jax.ShapeDtypeStruct((NC * L * G, D), jnp.float32),
                  jax.ShapeDtypeStruct((NC * G, D), jnp.float32)),
        mesh=mesh,
        scratch_types=[
            pltpu.VMEM_SHARED((G, D), jnp.float32),
            pltpu.VMEM_SHARED((G, D), jnp.float32),
            pltpu.VMEM_SHARED((G, D), jnp.float32),
            pltpu.VMEM_SHARED((G, D), jnp.float32),
            pltpu.VMEM_SHARED((G, D), jnp.float32),
            pltpu.VMEM((PK,), jnp.int32),
            pltpu.VMEM((PK, D), jnp.float32),
            pltpu.VMEM((PREM,), jnp.int32),
            pltpu.VMEM((PREM, D), jnp.float32),
            pltpu.VMEM((GPT, D), jnp.float32),
            pltpu.VMEM((PK, D), jnp.float32),
            pltpu.SemaphoreType.DMA,
        ],
    )


def _layer_body(h_ref, a0_ref, a1_ref, dinv_ref, w1_ref, b1_ref,
                g_ref, be_ref, w2_ref, b2_ref, o_ref):
    z = h_ref[...] + (a0_ref[...] + a1_ref[...]) * dinv_ref[...]
    z = jnp.dot(z, w1_ref[...], preferred_element_type=jnp.float32) + b1_ref[...]
    mu = jnp.mean(z, axis=0, keepdims=True)
    var = jnp.mean(jnp.square(z - mu), axis=0, keepdims=True)
    z = (z - mu) * jax.lax.rsqrt(var + 1e-5) * g_ref[...] + be_ref[...]
    z = jnp.maximum(z, 0.0)
    z = jnp.dot(z, w2_ref[...], preferred_element_type=jnp.float32) + b2_ref[...]
    o_ref[...] = jnp.maximum(z, 0.0)


def _tc_layer(h, a0, a1, dinv, w1, b1, gamma, beta, w2, b2):
    return pl.pallas_call(
        _layer_body,
        out_shape=jax.ShapeDtypeStruct((N, H), jnp.float32),
    )(h, a0, a1, dinv, w1, b1, gamma, beta, w2, b2)


def _dinv_body(d0_ref, d1_ref, o_ref):
    o_ref[...] = 1.0 / jnp.maximum(d0_ref[...] + d1_ref[...], 1.0)


def _tc_dinv(d0, d1):
    return pl.pallas_call(
        _dinv_body,
        out_shape=jax.ShapeDtypeStruct((N, 1), jnp.float32),
    )(d0, d1)


def _fc_body(pool_ref, cnt_ref, fcw_ref, fcb_ref, o_ref):
    cnt = jnp.maximum(cnt_ref[0:G, 0:1] + cnt_ref[G:2 * G, 0:1], 1.0)
    acc = jnp.zeros((G, C), jnp.float32) + fcb_ref[...]
    for l in range(L):
        p = (pool_ref[l * G:(l + 1) * G, :]
             + pool_ref[(L + l) * G:(L + l + 1) * G, :]) / cnt
        acc = acc + jnp.dot(p, fcw_ref[l * H:(l + 1) * H, :],
                            preferred_element_type=jnp.float32)
    m = jnp.max(acc, axis=-1, keepdims=True)
    sh = acc - m
    o_ref[...] = sh - jnp.log(jnp.sum(jnp.exp(sh), axis=-1, keepdims=True))


def _tc_fc(pool, cnt, fcw, fcb):
    return pl.pallas_call(
        _fc_body,
        out_shape=jax.ShapeDtypeStruct((G, C), jnp.float32),
    )(pool, cnt, fcw, fcb)


def kernel(x, edge_index, edge_attr, batch, W1, b1, gamma, beta, W2, b2,
           fcW, fcb):
    src = edge_index[0]
    dst = edge_index[1]
    pad = E_PAD - E
    src_p = jnp.concatenate([src, jnp.zeros((pad,), jnp.int32)])
    dst2 = jnp.concatenate([dst, jnp.full((pad,), N, jnp.int32)]).reshape(
        CH_TOT, K)
    agg = _make_agg()
    deg_k = _make_deg()
    pool_k = _make_pool()

    degf = deg_k(dst2)
    dinv = _tc_dinv(degf[:N, 0:1], degf[N:, 0:1])

    h = x
    hs = []
    for i in range(L):
        accf = agg(h, src_p, dst2)
        h = _tc_layer(h, accf[:N], accf[N:], dinv,
                      W1[i], b1[i].reshape(1, 2 * H),
                      gamma[i].reshape(1, 2 * H), beta[i].reshape(1, 2 * H),
                      W2[i], b2[i].reshape(1, H))
        hs.append(h)

    pool, cnt = pool_k(hs[0], hs[1], hs[2], hs[3], batch)
    return _tc_fc(pool, cnt, fcW, fcb.reshape(1, C))


# speedup vs baseline: 1.7816x; 1.7816x over previous
import functools

import jax
import jax.numpy as jnp
from jax import lax
from jax.experimental import pallas as pl
from jax.experimental.pallas import tpu as pltpu
from jax.experimental.pallas import tpu_sc as plsc

N = 10000
E = 320000
D = 128
H = 128
L = 4
C = 32
G = 256

NC = 2
NS = 16
EPC = E // NC
EPT = EPC // NS
K = 128
NFULL = EPT // K
TAIL = EPT - NFULL * K
ZB = 624
ZR = 16
ZCNT = ZB // ZR

CPT = 79
EPT_PAD = CPT * K
PAD_PT = EPT_PAD - EPT
NW = NC * NS

PR_T = (N // NC) // NS
PK = 104
PNC = PR_T // PK
PREM = N // NC - NS * PR_T
GPT = G // NS


def _zero_vmem(ref, rows):
    zero16 = jnp.zeros((16,), jnp.float32)
    for r in range(rows):
        for q in range(D // 16):
            ref[r, pl.ds(q * 16, 16)] = zero16


def _fill_ones(ref, rows):
    one16 = jnp.full((16,), 1.0, jnp.float32)
    for r in range(rows):
        for q in range(D // 16):
            ref[r, pl.ds(q * 16, 16)] = one16


ZT = 16


def _zero_acc(acc, zbuf, s):
    rbase = s * ZB

    @pl.loop(0, ZCNT)
    def _(i):
        pltpu.sync_copy(zbuf, acc.at[pl.ds(rbase + i * ZR, ZR), :])

    @pl.when(s == NS - 1)
    def _():
        pltpu.sync_copy(zbuf.at[pl.ds(0, ZT), :], acc.at[pl.ds(N - ZT, ZT), :])


def _write_acc(acc, out_hbm, c, s):
    rbase = s * ZB
    pltpu.sync_copy(acc.at[pl.ds(rbase, ZB), :],
                    out_hbm.at[pl.ds(c * N + rbase, ZB), :])

    @pl.when(s == NS - 1)
    def _():
        pltpu.sync_copy(acc.at[pl.ds(N - ZT, ZT), :],
                        out_hbm.at[pl.ds(c * N + N - ZT, ZT), :])


def _agg_body(h_hbm, src_hbm, dst3_hbm, out_hbm,
              acc, didx2, r0, r1, sb0, sb1, zbuf,
              g0, g1, i0, i1):
    c = lax.axis_index("c")
    s = lax.axis_index("s")
    rs = (r0, r1)
    sbs = (sb0, sb1)
    gsems = (g0, g1)
    isems = (i0, i1)

    _zero_vmem(zbuf, ZR)
    _zero_acc(acc, zbuf, s)

    w = c * NS + s
    pltpu.sync_copy(dst3_hbm.at[w], didx2)
    ebase = w * EPT_PAD

    plsc.subcore_barrier()

    def drain_g(b):
        pltpu.make_async_copy(h_hbm.at[sbs[b]], rs[b], gsems[b]).wait()

    def drain_i(b):
        pltpu.make_async_copy(src_hbm.at[pl.ds(0, K)], sbs[b],
                              isems[b]).wait()

    def load_idx(i, b):
        pltpu.async_copy(src_hbm.at[pl.ds(ebase + i * K, K)], sbs[b],
                         isems[b])

    pltpu.sync_copy(src_hbm.at[pl.ds(ebase, K)], sb0)
    load_idx(1, 1)
    pltpu.async_copy(h_hbm.at[sb0], r0, g0)

    def step(i, b, issue_idx):
        nb = 1 - b
        drain_g(b)
        if issue_idx:
            load_idx(i + 2, b)
        drain_i(nb)
        pltpu.async_copy(h_hbm.at[sbs[nb]], rs[nb], gsems[nb])
        pltpu.sync_copy(rs[b], acc.at[didx2.at[i]], add=True)

    @pl.loop(0, (CPT - 3) // 2)
    def _(j):
        i = j * 2
        step(i, 0, True)
        step(i + 1, 1, True)

    step(CPT - 3, 0, True)
    step(CPT - 2, 1, False)
    drain_g(0)
    pltpu.sync_copy(r0, acc.at[didx2.at[CPT - 1]], add=True)

    plsc.subcore_barrier()
    _write_acc(acc, out_hbm, c, s)


@functools.lru_cache(maxsize=None)
def _make_agg():
    mesh = plsc.VectorSubcoreMesh(core_axis_name="c", subcore_axis_name="s")
    return pl.kernel(
        _agg_body,
        out_type=jax.ShapeDtypeStruct((NC * N, D), jnp.float32),
        mesh=mesh,
        scratch_types=[
            pltpu.VMEM_SHARED((N + 8, D), jnp.float32),
            pltpu.VMEM((CPT, K), jnp.int32),
            pltpu.VMEM((K, D), jnp.float32),
            pltpu.VMEM((K, D), jnp.float32),
            pltpu.VMEM((K,), jnp.int32),
            pltpu.VMEM((K,), jnp.int32),
            pltpu.VMEM((ZR, D), jnp.float32),
            pltpu.SemaphoreType.DMA,
            pltpu.SemaphoreType.DMA,
            pltpu.SemaphoreType.DMA,
            pltpu.SemaphoreType.DMA,
        ],
    )


def _deg_body(dst3_hbm, out_hbm, acc, didx2, ones_v, zbuf, sem):
    c = lax.axis_index("c")
    s = lax.axis_index("s")

    _zero_vmem(zbuf, ZR)
    _fill_ones(ones_v, K)
    _zero_acc(acc, zbuf, s)

    pltpu.sync_copy(dst3_hbm.at[c * NS + s], didx2)

    plsc.subcore_barrier()

    @pl.loop(0, CPT)
    def _(i):
        pltpu.sync_copy(ones_v, acc.at[didx2.at[i]], add=True)

    plsc.subcore_barrier()
    _write_acc(acc, out_hbm, c, s)


@functools.lru_cache(maxsize=None)
def _make_deg():
    mesh = plsc.VectorSubcoreMesh(core_axis_name="c", subcore_axis_name="s")
    return pl.kernel(
        _deg_body,
        out_type=jax.ShapeDtypeStruct((NC * N, D), jnp.float32),
        mesh=mesh,
        scratch_types=[
            pltpu.VMEM_SHARED((N + 8, D), jnp.float32),
            pltpu.VMEM((CPT, K), jnp.int32),
            pltpu.VMEM((K, D), jnp.float32),
            pltpu.VMEM((ZR, D), jnp.float32),
            pltpu.SemaphoreType.DMA,
        ],
    )


def _pool_body(h0_hbm, h1_hbm, h2_hbm, h3_hbm, batch_hbm, out_hbm, cnt_hbm,
               a0, a1, a2, a3, cacc, bidx, rbuf, bidx_t, rbuf_t,
               zbuf, ones_v, sem):
    c = lax.axis_index("c")
    s = lax.axis_index("s")
    accs = (a0, a1, a2, a3)
    hs = (h0_hbm, h1_hbm, h2_hbm, h3_hbm)

    _zero_vmem(zbuf, GPT)
    _fill_ones(ones_v, PK)

    gbase = s * GPT
    for a in accs:
        pltpu.sync_copy(zbuf, a.at[pl.ds(gbase, GPT), :])
    pltpu.sync_copy(zbuf, cacc.at[pl.ds(gbase, GPT), :])
    plsc.subcore_barrier()

    nbase = c * (N // NC) + s * PR_T

    @pl.loop(0, PNC)
    def _(i):
        off = pl.multiple_of(nbase + i * PK, 8)
        pltpu.sync_copy(batch_hbm.at[pl.ds(off, PK)], bidx)
        for h_hbm, a in zip(hs, accs):
            pltpu.sync_copy(h_hbm.at[pl.ds(off, PK), :], rbuf)
            pltpu.sync_copy(rbuf, a.at[bidx], add=True)
        pltpu.sync_copy(ones_v, cacc.at[bidx], add=True)

    @pl.when(s == NS - 1)
    def _():
        off = pl.multiple_of(c * (N // NC) + NS * PR_T, 8)
        pltpu.sync_copy(batch_hbm.at[pl.ds(off, PREM)], bidx_t)
        for h_hbm, a in zip(hs, accs):
            pltpu.sync_copy(h_hbm.at[pl.ds(off, PREM), :], rbuf_t)
            pltpu.sync_copy(rbuf_t, a.at[bidx_t], add=True)
        pltpu.sync_copy(ones_v.at[pl.ds(0, PREM), :], cacc.at[bidx_t],
                        add=True)

    plsc.subcore_barrier()

    for l, a in enumerate(accs):
        obase = (c * L + l) * G + gbase
        pltpu.sync_copy(a.at[pl.ds(gbase, GPT), :],
                        out_hbm.at[pl.ds(obase, GPT), :])
    pltpu.sync_copy(cacc.at[pl.ds(gbase, GPT), :],
                    cnt_hbm.at[pl.ds(c * G + gbase, GPT), :])


@functools.lru_cache(maxsize=None)
def _make_pool():
    mesh = plsc.VectorSubcoreMesh(core_axis_name="c", subcore_axis_name="s")
    return pl.kernel(
        _pool_body,
        out_type=(jax.ShapeDtypeStruct((NC * L * G, D), jnp.float32),
                  jax.ShapeDtypeStruct((NC * G, D), jnp.float32)),
        mesh=mesh,
        scratch_types=[
            pltpu.VMEM_SHARED((G, D), jnp.float32),
            pltpu.VMEM_SHARED((G, D), jnp.float32),
            pltpu.VMEM_SHARED((G, D), jnp.float32),
            pltpu.VMEM_SHARED((G, D), jnp.float32),
            pltpu.VMEM_SHARED((G, D), jnp.float32),
            pltpu.VMEM((PK,), jnp.int32),
            pltpu.VMEM((PK, D), jnp.float32),
            pltpu.VMEM((PREM,), jnp.int32),
            pltpu.VMEM((PREM, D), jnp.float32),
            pltpu.VMEM((GPT, D), jnp.float32),
            pltpu.VMEM((PK, D), jnp.float32),
            pltpu.SemaphoreType.DMA,
        ],
    )


def _layer_body(h_ref, a0_ref, a1_ref, dinv_ref, w1_ref, b1_ref,
                g_ref, be_ref, w2_ref, b2_ref, o_ref):
    z = h_ref[...] + (a0_ref[...] + a1_ref[...]) * dinv_ref[...]
    z = jnp.dot(z, w1_ref[...], preferred_element_type=jnp.float32) + b1_ref[...]
    mu = jnp.mean(z, axis=0, keepdims=True)
    var = jnp.mean(jnp.square(z - mu), axis=0, keepdims=True)
    z = (z - mu) * jax.lax.rsqrt(var + 1e-5) * g_ref[...] + be_ref[...]
    z = jnp.maximum(z, 0.0)
    z = jnp.dot(z, w2_ref[...], preferred_element_type=jnp.float32) + b2_ref[...]
    o_ref[...] = jnp.maximum(z, 0.0)


def _tc_layer(h, a0, a1, dinv, w1, b1, gamma, beta, w2, b2):
    return pl.pallas_call(
        _layer_body,
        out_shape=jax.ShapeDtypeStruct((N, H), jnp.float32),
    )(h, a0, a1, dinv, w1, b1, gamma, beta, w2, b2)


def _dinv_body(d0_ref, d1_ref, o_ref):
    o_ref[...] = 1.0 / jnp.maximum(d0_ref[...] + d1_ref[...], 1.0)


def _tc_dinv(d0, d1):
    return pl.pallas_call(
        _dinv_body,
        out_shape=jax.ShapeDtypeStruct((N, 1), jnp.float32),
    )(d0, d1)


def _fc_body(pool_ref, cnt_ref, fcw_ref, fcb_ref, o_ref):
    cnt = jnp.maximum(cnt_ref[0:G, 0:1] + cnt_ref[G:2 * G, 0:1], 1.0)
    acc = jnp.zeros((G, C), jnp.float32) + fcb_ref[...]
    for l in range(L):
        p = (pool_ref[l * G:(l + 1) * G, :]
             + pool_ref[(L + l) * G:(L + l + 1) * G, :]) / cnt
        acc = acc + jnp.dot(p, fcw_ref[l * H:(l + 1) * H, :],
                            preferred_element_type=jnp.float32)
    m = jnp.max(acc, axis=-1, keepdims=True)
    sh = acc - m
    o_ref[...] = sh - jnp.log(jnp.sum(jnp.exp(sh), axis=-1, keepdims=True))


def _tc_fc(pool, cnt, fcw, fcb):
    return pl.pallas_call(
        _fc_body,
        out_shape=jax.ShapeDtypeStruct((G, C), jnp.float32),
    )(pool, cnt, fcw, fcb)


def kernel(x, edge_index, edge_attr, batch, W1, b1, gamma, beta, W2, b2,
           fcW, fcb):
    src = edge_index[0]
    dst = edge_index[1]
    src_p = jnp.concatenate(
        [src.reshape(NW, EPT), jnp.zeros((NW, PAD_PT), jnp.int32)],
        axis=1).reshape(NW * EPT_PAD)
    dst3 = jnp.concatenate(
        [dst.reshape(NW, EPT), jnp.full((NW, PAD_PT), N, jnp.int32)],
        axis=1).reshape(NW, CPT, K)
    agg = _make_agg()
    deg_k = _make_deg()
    pool_k = _make_pool()

    degf = deg_k(dst3)
    dinv = _tc_dinv(degf[:N, 0:1], degf[N:, 0:1])

    h = x
    hs = []
    for i in range(L):
        accf = agg(h, src_p, dst3)
        h = _tc_layer(h, accf[:N], accf[N:], dinv,
                      W1[i], b1[i].reshape(1, 2 * H),
                      gamma[i].reshape(1, 2 * H), beta[i].reshape(1, 2 * H),
                      W2[i], b2[i].reshape(1, H))
        hs.append(h)

    pool, cnt = pool_k(hs[0], hs[1], hs[2], hs[3], batch)
    return _tc_fc(pool, cnt, fcW, fcb.reshape(1, C))


# speedup vs baseline: 2.9568x; 1.6596x over previous
import functools

import jax
import jax.numpy as jnp
from jax import lax
from jax.experimental import pallas as pl
from jax.experimental.pallas import tpu as pltpu
from jax.experimental.pallas import tpu_sc as plsc

N = 10000
E = 320000
D = 128
H = 128
L = 4
C = 32
G = 256

NC = 2
NS = 16
EPC = E // NC
EPT = EPC // NS
K = 128
NFULL = EPT // K
TAIL = EPT - NFULL * K
ZB = 624
ZR = 16
ZCNT = ZB // ZR

CPT = 80
EPT_PAD = CPT * K
PAD_PT = EPT_PAD - EPT
NW = NC * NS

PR_T = (N // NC) // NS
PK = 104
PNC = PR_T // PK
PREM = N // NC - NS * PR_T
GPT = G // NS


def _zero_vmem(ref, rows):
    zero16 = jnp.zeros((16,), jnp.float32)
    for r in range(rows):
        for q in range(D // 16):
            ref[r, pl.ds(q * 16, 16)] = zero16


def _fill_ones(ref, rows):
    one16 = jnp.full((16,), 1.0, jnp.float32)
    for r in range(rows):
        for q in range(D // 16):
            ref[r, pl.ds(q * 16, 16)] = one16


ZT = 16


def _zero_acc(acc, zbuf, s):
    rbase = s * ZB

    @pl.loop(0, ZCNT)
    def _(i):
        pltpu.sync_copy(zbuf, acc.at[pl.ds(rbase + i * ZR, ZR), :])

    @pl.when(s == NS - 1)
    def _():
        pltpu.sync_copy(zbuf.at[pl.ds(0, ZT), :], acc.at[pl.ds(N - ZT, ZT), :])


def _write_acc(acc, out_hbm, c, s):
    rbase = s * ZB
    pltpu.sync_copy(acc.at[pl.ds(rbase, ZB), :],
                    out_hbm.at[pl.ds(c * N + rbase, ZB), :])

    @pl.when(s == NS - 1)
    def _():
        pltpu.sync_copy(acc.at[pl.ds(N - ZT, ZT), :],
                        out_hbm.at[pl.ds(c * N + N - ZT, ZT), :])


def _agg_body(h_hbm, src_hbm, dst3_hbm, out_hbm,
              acc, didx2, r0, r1, sb0, sb1, zbuf,
              g0, g1, i0, i1):
    c = lax.axis_index("c")
    s = lax.axis_index("s")
    rs = (r0, r1)
    sbs = (sb0, sb1)
    gsems = (g0, g1)
    isems = (i0, i1)

    _zero_vmem(zbuf, ZR)
    _zero_acc(acc, zbuf, s)

    w = c * NS + s
    pltpu.sync_copy(dst3_hbm.at[w], didx2)
    ebase = w * EPT_PAD

    plsc.subcore_barrier()

    def drain_g(b):
        pltpu.make_async_copy(h_hbm.at[sbs[b]], rs[b], gsems[b]).wait()

    def drain_i(b):
        pltpu.make_async_copy(src_hbm.at[pl.ds(0, K)], sbs[b],
                              isems[b]).wait()

    def load_idx(i, b):
        pltpu.async_copy(src_hbm.at[pl.ds(ebase + i * K, K)], sbs[b],
                         isems[b])

    pltpu.sync_copy(src_hbm.at[pl.ds(ebase, K)], sb0)
    load_idx(1, 1)
    pltpu.async_copy(h_hbm.at[sb0], r0, g0)

    def step(i, b, issue_idx):
        nb = 1 - b
        drain_g(b)
        if issue_idx:
            load_idx(i + 2, b)
        drain_i(nb)
        pltpu.async_copy(h_hbm.at[sbs[nb]], rs[nb], gsems[nb])
        pltpu.sync_copy(rs[b], acc.at[didx2.at[i]], add=True)

    @pl.loop(0, (CPT - 2) // 2)
    def _(j):
        i = j * 2
        step(i, 0, True)
        step(i + 1, 1, True)

    step(CPT - 2, 0, False)
    drain_g(1)
    pltpu.sync_copy(r1, acc.at[didx2.at[CPT - 1]], add=True)

    plsc.subcore_barrier()
    _write_acc(acc, out_hbm, c, s)


@functools.lru_cache(maxsize=None)
def _make_agg():
    mesh = plsc.VectorSubcoreMesh(core_axis_name="c", subcore_axis_name="s")
    return pl.kernel(
        _agg_body,
        out_type=jax.ShapeDtypeStruct((NC * N, D), jnp.float32),
        mesh=mesh,
        scratch_types=[
            pltpu.VMEM_SHARED((N + 8, D), jnp.float32),
            pltpu.VMEM((CPT, K), jnp.int32),
            pltpu.VMEM((K, D), jnp.float32),
            pltpu.VMEM((K, D), jnp.float32),
            pltpu.VMEM((K,), jnp.int32),
            pltpu.VMEM((K,), jnp.int32),
            pltpu.VMEM((ZR, D), jnp.float32),
            pltpu.SemaphoreType.DMA,
            pltpu.SemaphoreType.DMA,
            pltpu.SemaphoreType.DMA,
            pltpu.SemaphoreType.DMA,
        ],
    )


def _deg_body(dst3_hbm, out_hbm, acc, didx2, ones_v, zbuf, sem):
    c = lax.axis_index("c")
    s = lax.axis_index("s")

    _zero_vmem(zbuf, ZR)
    _fill_ones(ones_v, K)
    _zero_acc(acc, zbuf, s)

    pltpu.sync_copy(dst3_hbm.at[c * NS + s], didx2)

    plsc.subcore_barrier()

    @pl.loop(0, CPT)
    def _(i):
        pltpu.sync_copy(ones_v, acc.at[didx2.at[i]], add=True)

    plsc.subcore_barrier()
    _write_acc(acc, out_hbm, c, s)


@functools.lru_cache(maxsize=None)
def _make_deg():
    mesh = plsc.VectorSubcoreMesh(core_axis_name="c", subcore_axis_name="s")
    return pl.kernel(
        _deg_body,
        out_type=jax.ShapeDtypeStruct((NC * N, D), jnp.float32),
        mesh=mesh,
        scratch_types=[
            pltpu.VMEM_SHARED((N + 8, D), jnp.float32),
            pltpu.VMEM((CPT, K), jnp.int32),
            pltpu.VMEM((K, D), jnp.float32),
            pltpu.VMEM((ZR, D), jnp.float32),
            pltpu.SemaphoreType.DMA,
        ],
    )


def _pool_body(h0_hbm, h1_hbm, h2_hbm, h3_hbm, batch_hbm, out_hbm, cnt_hbm,
               a0, a1, a2, a3, cacc, bidx, rbuf, bidx_t, rbuf_t,
               zbuf, ones_v, sem):
    c = lax.axis_index("c")
    s = lax.axis_index("s")
    accs = (a0, a1, a2, a3)
    hs = (h0_hbm, h1_hbm, h2_hbm, h3_hbm)

    _zero_vmem(zbuf, GPT)
    _fill_ones(ones_v, PK)

    gbase = s * GPT
    for a in accs:
        pltpu.sync_copy(zbuf, a.at[pl.ds(gbase, GPT), :])
    pltpu.sync_copy(zbuf, cacc.at[pl.ds(gbase, GPT), :])
    plsc.subcore_barrier()

    nbase = c * (N // NC) + s * PR_T

    @pl.loop(0, PNC)
    def _(i):
        off = pl.multiple_of(nbase + i * PK, 8)
        pltpu.sync_copy(batch_hbm.at[pl.ds(off, PK)], bidx)
        for h_hbm, a in zip(hs, accs):
            pltpu.sync_copy(h_hbm.at[pl.ds(off, PK), :], rbuf)
            pltpu.sync_copy(rbuf, a.at[bidx], add=True)
        pltpu.sync_copy(ones_v, cacc.at[bidx], add=True)

    @pl.when(s == NS - 1)
    def _():
        off = pl.multiple_of(c * (N // NC) + NS * PR_T, 8)
        pltpu.sync_copy(batch_hbm.at[pl.ds(off, PREM)], bidx_t)
        for h_hbm, a in zip(hs, accs):
            pltpu.sync_copy(h_hbm.at[pl.ds(off, PREM), :], rbuf_t)
            pltpu.sync_copy(rbuf_t, a.at[bidx_t], add=True)
        pltpu.sync_copy(ones_v.at[pl.ds(0, PREM), :], cacc.at[bidx_t],
                        add=True)

    plsc.subcore_barrier()

    for l, a in enumerate(accs):
        obase = (c * L + l) * G + gbase
        pltpu.sync_copy(a.at[pl.ds(gbase, GPT), :],
                        out_hbm.at[pl.ds(obase, GPT), :])
    pltpu.sync_copy(cacc.at[pl.ds(gbase, GPT), :],
                    cnt_hbm.at[pl.ds(c * G + gbase, GPT), :])


@functools.lru_cache(maxsize=None)
def _make_pool():
    mesh = plsc.VectorSubcoreMesh(core_axis_name="c", subcore_axis_name="s")
    return pl.kernel(
        _pool_body,
        out_type=(jax.ShapeDtypeStruct((NC * L * G, D), jnp.float32),
                  jax.ShapeDtypeStruct((NC * G, D), jnp.float32)),
        mesh=mesh,
        scratch_types=[
            pltpu.VMEM_SHARED((G, D), jnp.float32),
            pltpu.VMEM_SHARED((G, D), jnp.float32),
            pltpu.VMEM_SHARED((G, D), jnp.float32),
            pltpu.VMEM_SHARED((G, D), jnp.float32),
            pltpu.VMEM_SHARED((G, D), jnp.float32),
            pltpu.VMEM((PK,), jnp.int32),
            pltpu.VMEM((PK, D), jnp.float32),
            pltpu.VMEM((PREM,), jnp.int32),
            pltpu.VMEM((PREM, D), jnp.float32),
            pltpu.VMEM((GPT, D), jnp.float32),
            pltpu.VMEM((PK, D), jnp.float32),
            pltpu.SemaphoreType.DMA,
        ],
    )


def _layer_body(h_ref, a0_ref, a1_ref, dinv_ref, w1_ref, b1_ref,
                g_ref, be_ref, w2_ref, b2_ref, o_ref):
    z = h_ref[...] + (a0_ref[...] + a1_ref[...]) * dinv_ref[...]
    z = jnp.dot(z, w1_ref[...], preferred_element_type=jnp.float32) + b1_ref[...]
    mu = jnp.mean(z, axis=0, keepdims=True)
    var = jnp.mean(jnp.square(z - mu), axis=0, keepdims=True)
    z = (z - mu) * jax.lax.rsqrt(var + 1e-5) * g_ref[...] + be_ref[...]
    z = jnp.maximum(z, 0.0)
    z = jnp.dot(z, w2_ref[...], preferred_element_type=jnp.float32) + b2_ref[...]
    o_ref[...] = jnp.maximum(z, 0.0)


def _tc_layer(h, a0, a1, dinv, w1, b1, gamma, beta, w2, b2):
    return pl.pallas_call(
        _layer_body,
        out_shape=jax.ShapeDtypeStruct((N, H), jnp.float32),
    )(h, a0, a1, dinv, w1, b1, gamma, beta, w2, b2)


def _dinv_body(d0_ref, d1_ref, o_ref):
    o_ref[...] = 1.0 / jnp.maximum(d0_ref[...] + d1_ref[...], 1.0)


def _tc_dinv(d0, d1):
    return pl.pallas_call(
        _dinv_body,
        out_shape=jax.ShapeDtypeStruct((N, 1), jnp.float32),
    )(d0, d1)


def _fc_body(pool_ref, cnt_ref, fcw_ref, fcb_ref, o_ref):
    cnt = jnp.maximum(cnt_ref[0:G, 0:1] + cnt_ref[G:2 * G, 0:1], 1.0)
    acc = jnp.zeros((G, C), jnp.float32) + fcb_ref[...]
    for l in range(L):
        p = (pool_ref[l * G:(l + 1) * G, :]
             + pool_ref[(L + l) * G:(L + l + 1) * G, :]) / cnt
        acc = acc + jnp.dot(p, fcw_ref[l * H:(l + 1) * H, :],
                            preferred_element_type=jnp.float32)
    m = jnp.max(acc, axis=-1, keepdims=True)
    sh = acc - m
    o_ref[...] = sh - jnp.log(jnp.sum(jnp.exp(sh), axis=-1, keepdims=True))


def _tc_fc(pool, cnt, fcw, fcb):
    return pl.pallas_call(
        _fc_body,
        out_shape=jax.ShapeDtypeStruct((G, C), jnp.float32),
    )(pool, cnt, fcw, fcb)


def kernel(x, edge_index, edge_attr, batch, W1, b1, gamma, beta, W2, b2,
           fcW, fcb):
    src = edge_index[0]
    dst = edge_index[1]
    pad_iota = jnp.arange(PAD_PT, dtype=jnp.int32)
    src_pad = jnp.broadcast_to(pad_iota % 64, (NW, PAD_PT))
    dst_pad = jnp.broadcast_to(N + (pad_iota % 8), (NW, PAD_PT))
    src_p = jnp.concatenate(
        [src.reshape(NW, EPT), src_pad], axis=1).reshape(NW * EPT_PAD)
    dst3 = jnp.concatenate(
        [dst.reshape(NW, EPT), dst_pad], axis=1).reshape(NW, CPT, K)
    agg = _make_agg()
    deg_k = _make_deg()
    pool_k = _make_pool()

    degf = deg_k(dst3)
    dinv = _tc_dinv(degf[:N, 0:1], degf[N:, 0:1])

    h = x
    hs = []
    for i in range(L):
        accf = agg(h, src_p, dst3)
        h = _tc_layer(h, accf[:N], accf[N:], dinv,
                      W1[i], b1[i].reshape(1, 2 * H),
                      gamma[i].reshape(1, 2 * H), beta[i].reshape(1, 2 * H),
                      W2[i], b2[i].reshape(1, H))
        hs.append(h)

    pool, cnt = pool_k(hs[0], hs[1], hs[2], hs[3], batch)
    return _tc_fc(pool, cnt, fcW, fcb.reshape(1, C))


# speedup vs baseline: 2.9615x; 1.0016x over previous
import functools

import jax
import jax.numpy as jnp
from jax import lax
from jax.experimental import pallas as pl
from jax.experimental.pallas import tpu as pltpu
from jax.experimental.pallas import tpu_sc as plsc

N = 10000
E = 320000
D = 128
H = 128
L = 4
C = 32
G = 256

NC = 2
NS = 16
EPC = E // NC
EPT = EPC // NS
K = 128
NFULL = EPT // K
TAIL = EPT - NFULL * K
ZB = 624
ZR = 16
ZCNT = ZB // ZR

CPT = 80
EPT_PAD = CPT * K
PAD_PT = EPT_PAD - EPT
NW = NC * NS

PR_T = (N // NC) // NS
PK = 104
PNC = PR_T // PK
PREM = N // NC - NS * PR_T
GPT = G // NS


def _zero_vmem(ref, rows):
    zero16 = jnp.zeros((16,), jnp.float32)
    for r in range(rows):
        for q in range(D // 16):
            ref[r, pl.ds(q * 16, 16)] = zero16


def _fill_ones(ref, rows):
    one16 = jnp.full((16,), 1.0, jnp.float32)
    for r in range(rows):
        for q in range(D // 16):
            ref[r, pl.ds(q * 16, 16)] = one16


ZT = 16


def _zero_acc(acc, zbuf, s):
    rbase = s * ZB

    @pl.loop(0, ZCNT)
    def _(i):
        pltpu.sync_copy(zbuf, acc.at[pl.ds(rbase + i * ZR, ZR), :])

    @pl.when(s == NS - 1)
    def _():
        pltpu.sync_copy(zbuf.at[pl.ds(0, ZT), :], acc.at[pl.ds(N - ZT, ZT), :])


def _write_acc(acc, out_hbm, c, s):
    rbase = s * ZB
    pltpu.sync_copy(acc.at[pl.ds(rbase, ZB), :],
                    out_hbm.at[pl.ds(c * N + rbase, ZB), :])

    @pl.when(s == NS - 1)
    def _():
        pltpu.sync_copy(acc.at[pl.ds(N - ZT, ZT), :],
                        out_hbm.at[pl.ds(c * N + N - ZT, ZT), :])


def _agg_body(h_hbm, src_hbm, dst3_hbm, out_hbm,
              acc, didx2, r0, r1, sb0, sb1, zbuf,
              g0, g1, i0, i1, s0, s1):
    c = lax.axis_index("c")
    s = lax.axis_index("s")
    rs = (r0, r1)
    sbs = (sb0, sb1)
    gsems = (g0, g1)
    isems = (i0, i1)
    ssems = (s0, s1)

    _zero_vmem(zbuf, ZR)
    _zero_acc(acc, zbuf, s)

    w = c * NS + s
    pltpu.sync_copy(dst3_hbm.at[w], didx2)
    ebase = w * EPT_PAD

    plsc.subcore_barrier()

    def drain_g(b):
        pltpu.make_async_copy(h_hbm.at[sbs[b]], rs[b], gsems[b]).wait()

    def drain_i(b):
        pltpu.make_async_copy(src_hbm.at[pl.ds(0, K)], sbs[b],
                              isems[b]).wait()

    def drain_s(b):
        pltpu.make_async_copy(rs[b], acc.at[didx2.at[0]], ssems[b]).wait()

    def load_idx(i, b):
        pltpu.async_copy(src_hbm.at[pl.ds(ebase + i * K, K)], sbs[b],
                         isems[b])

    pltpu.sync_copy(src_hbm.at[pl.ds(ebase, K)], sb0)
    load_idx(1, 1)
    pltpu.async_copy(h_hbm.at[sb0], r0, g0)

    def step(i, b, issue_idx, wait_prev_scatter=True):
        nb = 1 - b
        drain_g(b)
        pltpu.async_copy(rs[b], acc.at[didx2.at[i]], ssems[b],
                         add=True)
        if issue_idx:
            load_idx(i + 2, b)
        drain_i(nb)
        if wait_prev_scatter:
            drain_s(nb)
        pltpu.async_copy(h_hbm.at[sbs[nb]], rs[nb], gsems[nb])

    step(0, 0, True, wait_prev_scatter=False)

    @pl.loop(0, (CPT - 4) // 2)
    def _(j):
        i = 1 + j * 2
        step(i, 1, True)
        step(i + 1, 0, True)

    step(CPT - 3, 1, True)
    step(CPT - 2, 0, False)
    drain_g(1)
    pltpu.async_copy(r1, acc.at[didx2.at[CPT - 1]], ssems[1], add=True)
    drain_s(0)
    drain_s(1)

    plsc.subcore_barrier()
    _write_acc(acc, out_hbm, c, s)


@functools.lru_cache(maxsize=None)
def _make_agg():
    mesh = plsc.VectorSubcoreMesh(core_axis_name="c", subcore_axis_name="s")
    return pl.kernel(
        _agg_body,
        out_type=jax.ShapeDtypeStruct((NC * N, D), jnp.float32),
        mesh=mesh,
        scratch_types=[
            pltpu.VMEM_SHARED((N + 8, D), jnp.float32),
            pltpu.VMEM((CPT, K), jnp.int32),
            pltpu.VMEM((K, D), jnp.float32),
            pltpu.VMEM((K, D), jnp.float32),
            pltpu.VMEM((K,), jnp.int32),
            pltpu.VMEM((K,), jnp.int32),
            pltpu.VMEM((ZR, D), jnp.float32),
            pltpu.SemaphoreType.DMA,
            pltpu.SemaphoreType.DMA,
            pltpu.SemaphoreType.DMA,
            pltpu.SemaphoreType.DMA,
            pltpu.SemaphoreType.DMA,
            pltpu.SemaphoreType.DMA,
        ],
    )


def _deg_body(dst3_hbm, out_hbm, acc, didx2, ones_v, zbuf, sem):
    c = lax.axis_index("c")
    s = lax.axis_index("s")

    _zero_vmem(zbuf, ZR)
    _fill_ones(ones_v, K)
    _zero_acc(acc, zbuf, s)

    pltpu.sync_copy(dst3_hbm.at[c * NS + s], didx2)

    plsc.subcore_barrier()

    @pl.loop(0, CPT)
    def _(i):
        pltpu.sync_copy(ones_v, acc.at[didx2.at[i]], add=True)

    plsc.subcore_barrier()
    _write_acc(acc, out_hbm, c, s)


@functools.lru_cache(maxsize=None)
def _make_deg():
    mesh = plsc.VectorSubcoreMesh(core_axis_name="c", subcore_axis_name="s")
    return pl.kernel(
        _deg_body,
        out_type=jax.ShapeDtypeStruct((NC * N, D), jnp.float32),
        mesh=mesh,
        scratch_types=[
            pltpu.VMEM_SHARED((N + 8, D), jnp.float32),
            pltpu.VMEM((CPT, K), jnp.int32),
            pltpu.VMEM((K, D), jnp.float32),
            pltpu.VMEM((ZR, D), jnp.float32),
            pltpu.SemaphoreType.DMA,
        ],
    )


def _pool_body(h0_hbm, h1_hbm, h2_hbm, h3_hbm, batch_hbm, out_hbm, cnt_hbm,
               a0, a1, a2, a3, cacc, bidx, rbuf, bidx_t, rbuf_t,
               zbuf, ones_v, sem):
    c = lax.axis_index("c")
    s = lax.axis_index("s")
    accs = (a0, a1, a2, a3)
    hs = (h0_hbm, h1_hbm, h2_hbm, h3_hbm)

    _zero_vmem(zbuf, GPT)
    _fill_ones(ones_v, PK)

    gbase = s * GPT
    for a in accs:
        pltpu.sync_copy(zbuf, a.at[pl.ds(gbase, GPT), :])
    pltpu.sync_copy(zbuf, cacc.at[pl.ds(gbase, GPT), :])
    plsc.subcore_barrier()

    nbase = c * (N // NC) + s * PR_T

    @pl.loop(0, PNC)
    def _(i):
        off = pl.multiple_of(nbase + i * PK, 8)
        pltpu.sync_copy(batch_hbm.at[pl.ds(off, PK)], bidx)
        for h_hbm, a in zip(hs, accs):
            pltpu.sync_copy(h_hbm.at[pl.ds(off, PK), :], rbuf)
            pltpu.sync_copy(rbuf, a.at[bidx], add=True)
        pltpu.sync_copy(ones_v, cacc.at[bidx], add=True)

    @pl.when(s == NS - 1)
    def _():
        off = pl.multiple_of(c * (N // NC) + NS * PR_T, 8)
        pltpu.sync_copy(batch_hbm.at[pl.ds(off, PREM)], bidx_t)
        for h_hbm, a in zip(hs, accs):
            pltpu.sync_copy(h_hbm.at[pl.ds(off, PREM), :], rbuf_t)
            pltpu.sync_copy(rbuf_t, a.at[bidx_t], add=True)
        pltpu.sync_copy(ones_v.at[pl.ds(0, PREM), :], cacc.at[bidx_t],
                        add=True)

    plsc.subcore_barrier()

    for l, a in enumerate(accs):
        obase = (c * L + l) * G + gbase
        pltpu.sync_copy(a.at[pl.ds(gbase, GPT), :],
                        out_hbm.at[pl.ds(obase, GPT), :])
    pltpu.sync_copy(cacc.at[pl.ds(gbase, GPT), :],
                    cnt_hbm.at[pl.ds(c * G + gbase, GPT), :])


@functools.lru_cache(maxsize=None)
def _make_pool():
    mesh = plsc.VectorSubcoreMesh(core_axis_name="c", subcore_axis_name="s")
    return pl.kernel(
        _pool_body,
        out_type=(jax.ShapeDtypeStruct((NC * L * G, D), jnp.float32),
                  jax.ShapeDtypeStruct((NC * G, D), jnp.float32)),
        mesh=mesh,
        scratch_types=[
            pltpu.VMEM_SHARED((G, D), jnp.float32),
            pltpu.VMEM_SHARED((G, D), jnp.float32),
            pltpu.VMEM_SHARED((G, D), jnp.float32),
            pltpu.VMEM_SHARED((G, D), jnp.float32),
            pltpu.VMEM_SHARED((G, D), jnp.float32),
            pltpu.VMEM((PK,), jnp.int32),
            pltpu.VMEM((PK, D), jnp.float32),
            pltpu.VMEM((PREM,), jnp.int32),
            pltpu.VMEM((PREM, D), jnp.float32),
            pltpu.VMEM((GPT, D), jnp.float32),
            pltpu.VMEM((PK, D), jnp.float32),
            pltpu.SemaphoreType.DMA,
        ],
    )


def _layer_body(h_ref, a0_ref, a1_ref, dinv_ref, w1_ref, b1_ref,
                g_ref, be_ref, w2_ref, b2_ref, o_ref):
    z = h_ref[...] + (a0_ref[...] + a1_ref[...]) * dinv_ref[...]
    z = jnp.dot(z, w1_ref[...], preferred_element_type=jnp.float32) + b1_ref[...]
    mu = jnp.mean(z, axis=0, keepdims=True)
    var = jnp.mean(jnp.square(z - mu), axis=0, keepdims=True)
    z = (z - mu) * jax.lax.rsqrt(var + 1e-5) * g_ref[...] + be_ref[...]
    z = jnp.maximum(z, 0.0)
    z = jnp.dot(z, w2_ref[...], preferred_element_type=jnp.float32) + b2_ref[...]
    o_ref[...] = jnp.maximum(z, 0.0)


def _tc_layer(h, a0, a1, dinv, w1, b1, gamma, beta, w2, b2):
    return pl.pallas_call(
        _layer_body,
        out_shape=jax.ShapeDtypeStruct((N, H), jnp.float32),
    )(h, a0, a1, dinv, w1, b1, gamma, beta, w2, b2)


def _dinv_body(d0_ref, d1_ref, o_ref):
    o_ref[...] = 1.0 / jnp.maximum(d0_ref[...] + d1_ref[...], 1.0)


def _tc_dinv(d0, d1):
    return pl.pallas_call(
        _dinv_body,
        out_shape=jax.ShapeDtypeStruct((N, 1), jnp.float32),
    )(d0, d1)


def _fc_body(pool_ref, cnt_ref, fcw_ref, fcb_ref, o_ref):
    cnt = jnp.maximum(cnt_ref[0:G, 0:1] + cnt_ref[G:2 * G, 0:1], 1.0)
    acc = jnp.zeros((G, C), jnp.float32) + fcb_ref[...]
    for l in range(L):
        p = (pool_ref[l * G:(l + 1) * G, :]
             + pool_ref[(L + l) * G:(L + l + 1) * G, :]) / cnt
        acc = acc + jnp.dot(p, fcw_ref[l * H:(l + 1) * H, :],
                            preferred_element_type=jnp.float32)
    m = jnp.max(acc, axis=-1, keepdims=True)
    sh = acc - m
    o_ref[...] = sh - jnp.log(jnp.sum(jnp.exp(sh), axis=-1, keepdims=True))


def _tc_fc(pool, cnt, fcw, fcb):
    return pl.pallas_call(
        _fc_body,
        out_shape=jax.ShapeDtypeStruct((G, C), jnp.float32),
    )(pool, cnt, fcw, fcb)


def kernel(x, edge_index, edge_attr, batch, W1, b1, gamma, beta, W2, b2,
           fcW, fcb):
    src = edge_index[0]
    dst = edge_index[1]
    pad_iota = jnp.arange(PAD_PT, dtype=jnp.int32)
    src_pad = jnp.broadcast_to(pad_iota % 64, (NW, PAD_PT))
    dst_pad = jnp.broadcast_to(N + (pad_iota % 8), (NW, PAD_PT))
    src_p = jnp.concatenate(
        [src.reshape(NW, EPT), src_pad], axis=1).reshape(NW * EPT_PAD)
    dst3 = jnp.concatenate(
        [dst.reshape(NW, EPT), dst_pad], axis=1).reshape(NW, CPT, K)
    agg = _make_agg()
    deg_k = _make_deg()
    pool_k = _make_pool()

    degf = deg_k(dst3)
    dinv = _tc_dinv(degf[:N, 0:1], degf[N:, 0:1])

    h = x
    hs = []
    for i in range(L):
        accf = agg(h, src_p, dst3)
        h = _tc_layer(h, accf[:N], accf[N:], dinv,
                      W1[i], b1[i].reshape(1, 2 * H),
                      gamma[i].reshape(1, 2 * H), beta[i].reshape(1, 2 * H),
                      W2[i], b2[i].reshape(1, H))
        hs.append(h)

    pool, cnt = pool_k(hs[0], hs[1], hs[2], hs[3], batch)
    return _tc_fc(pool, cnt, fcW, fcb.reshape(1, C))
